# baseline (speedup 1.0000x reference)
import functools

import jax
import jax.numpy as jnp
from jax import lax
from jax.experimental import pallas as pl
from jax.experimental.pallas import tpu as pltpu
from jax.experimental.pallas import tpu_sc as plsc

L = 16


def _pad16(n):
    return ((n + L - 1) // L) * L


@functools.partial(jax.jit, static_argnames=("B", "A", "E", "NS"))
def _repulsive_sc(elm, eng, edge_n, edge_i, edge_j, sod, eng_atm,
                  radius, *, B, A, E, NS):
    NW = 32
    BP = _pad16(B)
    NSP = _pad16(NS)
    EPW = E // NW
    CE = 2000
    NCH = EPW // CE
    NG = (A + L - 1) // L
    OWN = -(-B // (NW - 7))
    STG = 8
    SFULL = (B // STG) * STG
    TAILN = B - SFULL

    mesh = plsc.VectorSubcoreMesh(core_axis_name="c", subcore_axis_name="s")

    @functools.partial(
        pl.kernel,
        out_type=jax.ShapeDtypeStruct((NW, BP), jnp.float32),
        mesh=mesh,
        compiler_params=pltpu.CompilerParams(needs_layout_passes=False),
        scratch_types=[
            pltpu.VMEM((NSP,), jnp.float32),
            pltpu.VMEM((NSP,), jnp.float32),
            pltpu.VMEM((BP,), jnp.float32),
            pltpu.VMEM((BP,), jnp.float32),
            pltpu.VMEM((B * A,), jnp.float32),
            pltpu.VMEM((STG, A), jnp.int32),
            pltpu.VMEM((STG, A), jnp.int32),
            pltpu.VMEM((CE,), jnp.int32),
            pltpu.VMEM((CE,), jnp.int32),
            pltpu.VMEM((CE,), jnp.int32),
            pltpu.VMEM((CE,), jnp.float32),
            pltpu.SemaphoreType.DMA,
        ],
    )
    def k(elm_h, eng_h, en_h, ei_h, ej_h, sod_h, eatm_h, rad_h, out_h,
          rtbl_v, etbl_v, eng_v, res_v, ratm_v, bufa_v, bufb_v,
          n_v, i_v, j_v, sod_v, sem):
        wid = lax.axis_index("s") * 2 + lax.axis_index("c")
        iota = lax.iota(jnp.int32, L)
        lane0 = iota == 0
        zf = jnp.zeros((L,), jnp.float32)
        onesf = jnp.full((L,), 1.0, jnp.float32)

        def _stage(s0, buf_v, do):
            def full(_):
                cp = pltpu.make_async_copy(
                    elm_h.at[pl.ds(pl.multiple_of(s0, STG), STG)],
                    buf_v, sem)
                do(cp)
                return 0

            if TAILN == 0:
                full(0)
                return

            def short(_):
                cp = pltpu.make_async_copy(
                    elm_h.at[pl.ds(SFULL, TAILN)],
                    buf_v.at[pl.ds(0, TAILN)], sem)
                do(cp)
                return 0

            lax.cond(s0 < SFULL, full, short, 0)

        def stage_start(s0, buf_v):
            _stage(s0, buf_v, lambda cp: cp.start())

        def stage_wait(s0, buf_v):
            _stage(s0, buf_v, lambda cp: cp.wait())

        own_lo = jnp.minimum(wid * OWN, B)
        own_hi = jnp.minimum(own_lo + OWN, B)
        s0_own = (own_lo // STG) * STG
        base0 = pl.multiple_of(wid * EPW, 8)

        cps = [
            pltpu.async_copy(rad_h, rtbl_v.at[pl.ds(0, NS)], sem),
            pltpu.async_copy(eatm_h, etbl_v.at[pl.ds(0, NS)], sem),
            pltpu.async_copy(eng_h, eng_v.at[pl.ds(0, B)], sem),
            pltpu.async_copy(en_h.at[pl.ds(base0, L)],
                             n_v.at[pl.ds(0, L)], sem),
            pltpu.async_copy(en_h.at[pl.ds(base0 + EPW - L, L)],
                             n_v.at[pl.ds(L, L)], sem),
        ]
        stage_start(s0_own, bufa_v)
        for g in range(BP // L):
            res_v[pl.ds(g * L, L)] = zf
        for cp in cps:
            cp.wait()
        stage_wait(s0_own, bufa_v)

        nlo = jnp.min(n_v[pl.ds(0, L)])
        nhi = jnp.max(n_v[pl.ds(L, L)])
        s0_sl = (nlo // STG) * STG
        stage_start(s0_sl, bufb_v)

        def esum_one(buf_v, s0):
            def run(b, _):
                row = b - s0

                def grp(g, acc):
                    start = jnp.minimum(g * L, A - L)
                    v = buf_v[row, pl.ds(start, L)]
                    valid = (start + iota) >= g * L
                    e = plsc.load_gather(etbl_v, [v], mask=valid)
                    return acc + jnp.where(valid, e, zf)

                acc = lax.fori_loop(0, NG, grp, zf, unroll=7)
                tot = jnp.full((L,), jnp.sum(acc), jnp.float32)
                b16 = jnp.full((L,), b, jnp.int32)
                e16 = plsc.load_gather(eng_v, [b16])
                flag = jnp.where(e16 >= tot, onesf, zf)
                plsc.store_scatter(res_v, [b16], flag, mask=lane0)
                return 0

            return run

        lax.fori_loop(own_lo, own_hi, esum_one(bufa_v, s0_own), 0)

        stage_wait(s0_sl, bufb_v)

        def sl_cond(s0):
            return s0 <= nhi

        def sl_body(s0):
            lo = jnp.maximum(nlo, s0)
            hi = jnp.minimum(nhi + 1, s0 + STG)
            lax.fori_loop(lo, hi, esum_one(bufb_v, s0), 0)
            s0n = s0 + STG

            @pl.when(s0n <= nhi)
            def _():
                stage_start(s0n, bufb_v)
                stage_wait(s0n, bufb_v)

            return s0n

        lax.while_loop(sl_cond, sl_body, s0_sl)

        def range_min(rlo, rhi):
            nseg = (rhi - rlo) // L + 1
            rhi16 = jnp.full((L,), rhi, jnp.int32)

            def seg(s, m):
                probe = jnp.minimum(rlo + s * L + iota, rhi16)
                f = plsc.load_gather(res_v, [probe])
                return jnp.minimum(m, jnp.min(f))

            return lax.fori_loop(0, nseg, seg, jnp.float32(1.0))

        slice_done = range_min(nlo, nhi) > 0.0

        def edge_phase(_):
            def build_one(s0):
                def run(b, _):
                    row = b - s0

                    def grp(g, _):
                        start = jnp.minimum(g * L, A - L)
                        v = bufa_v[row, pl.ds(start, L)]
                        ratm_v[pl.ds(b * A + start, L)] = \
                            plsc.load_gather(rtbl_v, [v])
                        return 0

                    lax.fori_loop(0, NG, grp, 0, unroll=7)
                    return 0

                return run

            def bd_cond(s0):
                return s0 <= nhi

            def bd_body(s0):
                stage_start(s0, bufa_v)
                stage_wait(s0, bufa_v)
                lo = jnp.maximum(nlo, s0)
                hi = jnp.minimum(nhi + 1, s0 + STG)
                lax.fori_loop(lo, hi, build_one(s0), 0)
                return s0 + STG

            lax.while_loop(bd_cond, bd_body, (nlo // STG) * STG)

            def chunk_cond(carry):
                c, done = carry
                return jnp.logical_and(c < NCH, jnp.logical_not(done))

            def chunk_body(carry):
                c, done = carry
                base = pl.multiple_of(wid * EPW + c * CE, 8)
                pltpu.async_copy(en_h.at[pl.ds(base, CE)], n_v, sem).wait()
                cmin = jnp.min(n_v[pl.ds(0, L)])
                cmax = jnp.max(n_v[pl.ds(CE - L, L)])
                skippable = range_min(cmin, cmax) > 0.0

                def process(_):
                    cp2 = pltpu.async_copy(ei_h.at[pl.ds(base, CE)], i_v,
                                           sem)
                    cp3 = pltpu.async_copy(ej_h.at[pl.ds(base, CE)], j_v,
                                           sem)
                    cp4 = pltpu.async_copy(sod_h.at[pl.ds(base, CE)],
                                           sod_v, sem)
                    cp2.wait()
                    cp3.wait()
                    cp4.wait()

                    def grp(g, _):
                        n16 = n_v[pl.ds(g * L, L)]
                        i16 = i_v[pl.ds(g * L, L)]
                        j16 = j_v[pl.ds(g * L, L)]
                        s16 = sod_v[pl.ds(g * L, L)]
                        ri = plsc.load_gather(ratm_v, [n16 * A + i16])
                        rj = plsc.load_gather(ratm_v, [n16 * A + j16])
                        rs = ri + rj
                        plsc.store_scatter(res_v, [n16], onesf,
                                           mask=(rs * rs) >= s16)
                        return 0

                    lax.fori_loop(0, CE // L, grp, 0)
                    return range_min(nlo, nhi) > 0.0

                newdone = lax.cond(skippable, lambda _: done, process, 0)
                return (c + 1, newdone)

            lax.while_loop(chunk_cond, chunk_body,
                           (jnp.int32(0), jnp.bool_(False)))
            return 0

        lax.cond(slice_done, lambda _: 0, edge_phase, 0)

        pltpu.sync_copy(res_v, out_h.at[wid])

    return k(elm, eng, edge_n, edge_i, edge_j, sod, eng_atm, radius)


def kernel(elm, eng, edge_n, edge_i, edge_j, sod, eng_atm, radius):
    B, A = elm.shape
    E = edge_n.shape[0]
    NS = radius.shape[0]
    rows = _repulsive_sc(elm, eng, edge_n, edge_i, edge_j, sod,
                         eng_atm, radius, B=B, A=A, E=E, NS=NS)
    return jnp.max(rows, axis=0)[:B] > 0.5


# speedup vs baseline: 1.0500x; 1.0500x over previous
import functools

import jax
import jax.numpy as jnp
from jax import lax
from jax.experimental import pallas as pl
from jax.experimental.pallas import tpu as pltpu
from jax.experimental.pallas import tpu_sc as plsc

L = 16


def _pad16(n):
    return ((n + L - 1) // L) * L


@functools.partial(jax.jit, static_argnames=("B", "A", "E", "NS"))
def _repulsive_sc(elm_flat, eng, edge_n, edge_i, edge_j, sod, eng_atm,
                  radius, *, B, A, E, NS):
    NW = 32
    BP = _pad16(B)
    NSP = _pad16(NS)
    EPW = E // NW
    CE = 2000
    NCH = EPW // CE
    NG = (A + L - 1) // L
    OWN = -(-B // (NW - 7))
    STG = 8

    mesh = plsc.VectorSubcoreMesh(core_axis_name="c", subcore_axis_name="s")

    @functools.partial(
        pl.kernel,
        out_type=jax.ShapeDtypeStruct((NW, BP), jnp.float32),
        mesh=mesh,
        compiler_params=pltpu.CompilerParams(needs_layout_passes=False),
        scratch_types=[
            pltpu.VMEM((NSP,), jnp.float32),
            pltpu.VMEM((NSP,), jnp.float32),
            pltpu.VMEM((BP,), jnp.float32),
            pltpu.VMEM((BP,), jnp.float32),
            pltpu.VMEM((B * A + L,), jnp.float32),
            pltpu.VMEM((STG * A + 8,), jnp.int32),
            pltpu.VMEM((STG * A + 8,), jnp.int32),
            pltpu.VMEM((CE,), jnp.int32),
            pltpu.VMEM((CE,), jnp.int32),
            pltpu.VMEM((CE,), jnp.int32),
            pltpu.VMEM((CE,), jnp.float32),
            pltpu.SemaphoreType.DMA,
        ],
    )
    def k(elm_h, eng_h, en_h, ei_h, ej_h, sod_h, eatm_h, rad_h, out_h,
          rtbl_v, etbl_v, eng_v, res_v, ratm_v, bufa_v, bufb_v,
          n_v, i_v, j_v, sod_v, sem):
        wid = lax.axis_index("s") * 2 + lax.axis_index("c")
        iota = lax.iota(jnp.int32, L)
        lane0 = iota == 0
        zf = jnp.zeros((L,), jnp.float32)
        onesf = jnp.full((L,), 1.0, jnp.float32)

        own_lo = jnp.minimum(wid * OWN, B)
        own_hi = jnp.minimum(own_lo + OWN, B)
        s0_own = jnp.maximum(0, jnp.minimum(own_lo, B - STG))
        base0 = pl.multiple_of(wid * EPW, 8)

        cps = [
            pltpu.async_copy(rad_h, rtbl_v.at[pl.ds(0, NS)], sem),
            pltpu.async_copy(eatm_h, etbl_v.at[pl.ds(0, NS)], sem),
            pltpu.async_copy(eng_h, eng_v.at[pl.ds(0, B)], sem),
            pltpu.async_copy(en_h.at[pl.ds(base0, L)],
                             n_v.at[pl.ds(0, L)], sem),
            pltpu.async_copy(en_h.at[pl.ds(base0 + EPW - L, L)],
                             n_v.at[pl.ds(L, L)], sem),
            pltpu.async_copy(elm_h.at[pl.ds(s0_own * A, STG * A)],
                             bufa_v.at[pl.ds(0, STG * A)], sem),
        ]
        for g in range(BP // L):
            res_v[pl.ds(g * L, L)] = zf
        for cp in cps:
            cp.wait()

        nlo = jnp.min(n_v[pl.ds(0, L)])
        nhi = jnp.max(n_v[pl.ds(L, L)])
        s0_sl = jnp.maximum(0, jnp.minimum(nlo, B - STG))
        cp_sl = pltpu.async_copy(elm_h.at[pl.ds(s0_sl * A, STG * A)],
                                 bufb_v.at[pl.ds(0, STG * A)], sem)

        def esum_one(buf_v, s0):
            def run(b, _):
                off = (b - s0) * A

                def grp(g, acc):
                    v = buf_v[pl.ds(off + g * L, L)]
                    valid = (g * L + iota) < A
                    e = plsc.load_gather(etbl_v, [v], mask=valid)
                    return acc + jnp.where(valid, e, zf)

                acc = lax.fori_loop(0, NG, grp, zf, unroll=7)
                tot = jnp.full((L,), jnp.sum(acc), jnp.float32)
                b16 = jnp.full((L,), b, jnp.int32)
                e16 = plsc.load_gather(eng_v, [b16])
                flag = jnp.where(e16 >= tot, onesf, zf)
                plsc.store_scatter(res_v, [b16], flag, mask=lane0)
                return 0

            return run

        lax.fori_loop(own_lo, own_hi, esum_one(bufa_v, s0_own), 0)

        cp_sl.wait()

        def sl_cond(carry):
            lo, _ = carry
            return lo <= nhi

        def sl_body(carry):
            lo, s0 = carry
            hi_blk = jnp.minimum(lo + STG, nhi + 1)
            lax.fori_loop(lo, hi_blk, esum_one(bufb_v, s0), 0)
            nlo2 = lo + STG
            s02 = jnp.maximum(0, jnp.minimum(nlo2, B - STG))

            @pl.when(nlo2 <= nhi)
            def _():
                pltpu.sync_copy(elm_h.at[pl.ds(s02 * A, STG * A)],
                                bufb_v.at[pl.ds(0, STG * A)])

            return (nlo2, s02)

        lax.while_loop(sl_cond, sl_body, (nlo, s0_sl))

        def range_min(rlo, rhi):
            nseg = (rhi - rlo) // L + 1
            rhi16 = jnp.full((L,), rhi, jnp.int32)

            def seg(s, m):
                probe = jnp.minimum(rlo + s * L + iota, rhi16)
                f = plsc.load_gather(res_v, [probe])
                return jnp.minimum(m, jnp.min(f))

            return lax.fori_loop(0, nseg, seg, jnp.float32(1.0))

        slice_done = range_min(nlo, nhi) > 0.0

        def edge_phase(_):
            def build_one(b, _):
                pltpu.sync_copy(elm_h.at[pl.ds(b * A, A)],
                                bufa_v.at[pl.ds(0, A)])

                def grp(g, _):
                    v = bufa_v[pl.ds(g * L, L)]
                    ratm_v[pl.ds(b * A + g * L, L)] = \
                        plsc.load_gather(rtbl_v, [v])
                    return 0

                lax.fori_loop(0, NG, grp, 0, unroll=7)
                return 0

            lax.fori_loop(nlo, nhi + 1, build_one, 0)

            def chunk_cond(carry):
                c, done = carry
                return jnp.logical_and(c < NCH, jnp.logical_not(done))

            def chunk_body(carry):
                c, done = carry
                base = pl.multiple_of(wid * EPW + c * CE, 8)
                pltpu.async_copy(en_h.at[pl.ds(base, CE)], n_v, sem).wait()
                cmin = jnp.min(n_v[pl.ds(0, L)])
                cmax = jnp.max(n_v[pl.ds(CE - L, L)])
                skippable = range_min(cmin, cmax) > 0.0

                def process(_):
                    cp2 = pltpu.async_copy(ei_h.at[pl.ds(base, CE)], i_v,
                                           sem)
                    cp3 = pltpu.async_copy(ej_h.at[pl.ds(base, CE)], j_v,
                                           sem)
                    cp4 = pltpu.async_copy(sod_h.at[pl.ds(base, CE)],
                                           sod_v, sem)
                    cp2.wait()
                    cp3.wait()
                    cp4.wait()

                    def grp(g, _):
                        n16 = n_v[pl.ds(g * L, L)]
                        i16 = i_v[pl.ds(g * L, L)]
                        j16 = j_v[pl.ds(g * L, L)]
                        s16 = sod_v[pl.ds(g * L, L)]
                        ri = plsc.load_gather(ratm_v, [n16 * A + i16])
                        rj = plsc.load_gather(ratm_v, [n16 * A + j16])
                        rs = ri + rj
                        plsc.store_scatter(res_v, [n16], onesf,
                                           mask=(rs * rs) >= s16)
                        return 0

                    lax.fori_loop(0, CE // L, grp, 0)
                    return range_min(nlo, nhi) > 0.0

                newdone = lax.cond(skippable, lambda _: done, process, 0)
                return (c + 1, newdone)

            lax.while_loop(chunk_cond, chunk_body,
                           (jnp.int32(0), jnp.bool_(False)))
            return 0

        lax.cond(slice_done, lambda _: 0, edge_phase, 0)

        pltpu.sync_copy(res_v, out_h.at[wid])

    return k(elm_flat, eng, edge_n, edge_i, edge_j, sod, eng_atm, radius)


def kernel(elm, eng, edge_n, edge_i, edge_j, sod, eng_atm, radius):
    B, A = elm.shape
    E = edge_n.shape[0]
    NS = radius.shape[0]
    rows = _repulsive_sc(elm.reshape(-1), eng, edge_n, edge_i, edge_j, sod,
                         eng_atm, radius, B=B, A=A, E=E, NS=NS)
    return jnp.max(rows, axis=0)[:B] > 0.5
